# Initial kernel scaffold; baseline (speedup 1.0000x reference)
import jax
import jax.numpy as jnp
from jax.experimental import pallas as pl


def kernel(x, edge_index, W1, b1, W2, b2):
    raise NotImplementedError("write your pallas kernel here")


# speedup vs baseline: 51.5528x; 51.5528x over previous
import jax
import jax.numpy as jnp
from jax import lax
from jax.experimental import pallas as pl
from jax.experimental.pallas import tpu as pltpu
from jax.experimental.pallas import tpu_sc as plsc

N_NODES = 100000
N_EDGES = 6400000
BATCH = 4

NC = 2
NS = 16
NW = NC * NS

N_PAD = 100352
ROWS_PER_TILE = N_PAD // NS

CHUNK = 8192
E_PER_W = 204800
N_CHUNKS = E_PER_W // CHUNK
E_PAD = E_PER_W * NW

_mesh = plsc.VectorSubcoreMesh(core_axis_name="c", subcore_axis_name="s")
_sc_params = pltpu.CompilerParams(use_tc_tiling_on_sc=False)


def _deg_body(dst_hbm, degp_hbm, idx_v, ones_v, fill_v, deg_sh):
    c = lax.axis_index("c")
    s = lax.axis_index("s")
    w = c * NS + s
    rs = pl.ds(s * ROWS_PER_TILE, ROWS_PER_TILE)

    def fill_ones(i, carry):
        ones_v[pl.ds(i * 16, 16)] = jnp.full((16,), 1.0, jnp.float32)
        return carry

    lax.fori_loop(0, CHUNK // 16, fill_ones, 0)

    def fill_half(i, carry):
        fill_v[pl.ds(i * 16, 16)] = jnp.full((16,), 0.5, jnp.float32)
        return carry

    lax.fori_loop(0, ROWS_PER_TILE // 16, fill_half, 0)
    pltpu.sync_copy(fill_v, deg_sh.at[rs])
    plsc.subcore_barrier()

    base = w * E_PER_W

    def body(i, carry):
        off = pl.multiple_of(base + i * CHUNK, CHUNK)
        pltpu.sync_copy(dst_hbm.at[pl.ds(off, CHUNK)], idx_v)
        pltpu.sync_copy(ones_v, deg_sh.at[idx_v], add=True)
        return carry

    lax.fori_loop(0, N_CHUNKS, body, 0)
    plsc.subcore_barrier()
    pltpu.sync_copy(deg_sh.at[rs], degp_hbm.at[c].at[rs])


_deg_call = pl.kernel(
    _deg_body,
    out_type=jax.ShapeDtypeStruct((NC, N_PAD), jnp.float32),
    mesh=_mesh,
    compiler_params=_sc_params,
    scratch_types=[
        pltpu.VMEM((CHUNK,), jnp.int32),
        pltpu.VMEM((CHUNK,), jnp.float32),
        pltpu.VMEM((ROWS_PER_TILE,), jnp.float32),
        pltpu.VMEM_SHARED((N_PAD,), jnp.float32),
    ],
)


def _edge_body(src_hbm, dst_hbm, g0_hbm, g1_hbm, g2_hbm, g3_hbm,
               a0_hbm, a1_hbm, a2_hbm, a3_hbm,
               src_v, dst_v, msg_v, buf_v,
               g0_sh, g1_sh, g2_sh, g3_sh, ac0_sh, ac1_sh, ac2_sh, ac3_sh):
    c = lax.axis_index("c")
    s = lax.axis_index("s")
    w = c * NS + s
    rs = pl.ds(s * ROWS_PER_TILE, ROWS_PER_TILE)
    g_hbms = (g0_hbm, g1_hbm, g2_hbm, g3_hbm)
    a_hbms = (a0_hbm, a1_hbm, a2_hbm, a3_hbm)
    g_shs = (g0_sh, g1_sh, g2_sh, g3_sh)
    a_shs = (ac0_sh, ac1_sh, ac2_sh, ac3_sh)

    for b in range(BATCH):
        pltpu.sync_copy(g_hbms[b].at[rs], buf_v)
        pltpu.sync_copy(buf_v, g_shs[b].at[rs])

    def fill_zero(i, carry):
        buf_v[pl.ds(i * 16, 16)] = jnp.full((16,), 0.0, jnp.float32)
        return carry

    lax.fori_loop(0, ROWS_PER_TILE // 16, fill_zero, 0)
    for b in range(BATCH):
        pltpu.sync_copy(buf_v, a_shs[b].at[rs])
    plsc.subcore_barrier()

    base = w * E_PER_W

    def body(i, carry):
        off = pl.multiple_of(base + i * CHUNK, CHUNK)
        pltpu.sync_copy(src_hbm.at[pl.ds(off, CHUNK)], src_v)
        pltpu.sync_copy(dst_hbm.at[pl.ds(off, CHUNK)], dst_v)
        for b in range(BATCH):
            pltpu.sync_copy(g_shs[b].at[src_v], msg_v)
            pltpu.sync_copy(msg_v, a_shs[b].at[dst_v], add=True)
        return carry

    lax.fori_loop(0, N_CHUNKS, body, 0)
    plsc.subcore_barrier()
    for b in range(BATCH):
        pltpu.sync_copy(a_shs[b].at[rs], a_hbms[b].at[c].at[rs])


_edge_call = pl.kernel(
    _edge_body,
    out_type=tuple(jax.ShapeDtypeStruct((NC, N_PAD), jnp.float32)
                   for _ in range(BATCH)),
    mesh=_mesh,
    compiler_params=_sc_params,
    scratch_types=(
        [pltpu.VMEM((CHUNK,), jnp.int32)] * 2
        + [pltpu.VMEM((CHUNK,), jnp.float32)]
        + [pltpu.VMEM((ROWS_PER_TILE,), jnp.float32)]
        + [pltpu.VMEM_SHARED((N_PAD,), jnp.float32)] * 8
    ),
)


def _prep1_body(w1_ref, degp_ref, x4_ref, g1_ref, dis_ref):
    deg = degp_ref[0:1, :] + degp_ref[1:2, :]
    dis = lax.rsqrt(deg)
    dis_ref[...] = dis
    g1_ref[...] = x4_ref[...] * (dis * w1_ref[0:1, 0:1])


_prep1_call = pl.pallas_call(
    _prep1_body,
    out_shape=(
        jax.ShapeDtypeStruct((BATCH, N_PAD), jnp.float32),
        jax.ShapeDtypeStruct((1, N_PAD), jnp.float32),
    ),
)


def _prep2_body(w2_ref, b1_ref, at_ref, g1t_ref, dis_ref, g2_ref):
    a = at_ref[:, 0, :] + at_ref[:, 1, :]
    y1 = dis_ref[...] * (a + g1t_ref[...]) + b1_ref[0:1, 0:1]
    h = jnp.maximum(y1, 0.0)
    g2_ref[...] = h * (dis_ref[...] * w2_ref[0:1, 0:1])


_prep2_call = pl.pallas_call(
    _prep2_body,
    out_shape=jax.ShapeDtypeStruct((BATCH, N_PAD), jnp.float32),
)


def _out_body(b2_ref, at_ref, g2t_ref, dis_ref, y_ref):
    a = at_ref[:, 0, :] + at_ref[:, 1, :]
    y_ref[...] = dis_ref[...] * (a + g2t_ref[...]) + b2_ref[0:1, 0:1]


_out_call = pl.pallas_call(
    _out_body,
    out_shape=jax.ShapeDtypeStruct((BATCH, N_PAD), jnp.float32),
)


def kernel(x, edge_index, W1, b1, W2, b2):
    n = x.shape[1]
    e = edge_index.shape[1]
    e32 = edge_index.astype(jnp.int32)
    pad = jnp.full((2, E_PAD - e), n, dtype=jnp.int32)
    e32 = jnp.concatenate([e32, pad], axis=1)
    src = e32[0]
    dst = e32[1]

    x4 = jnp.pad(x[:, :, 0], ((0, 0), (0, N_PAD - n)))
    w1 = W1.reshape(1, 1)
    w2 = W2.reshape(1, 1)
    b1r = b1.reshape(1, 1)
    b2r = b2.reshape(1, 1)

    degp = _deg_call(dst)
    g1t, dis = _prep1_call(w1, degp, x4)
    acc1 = _edge_call(src, dst, g1t[0], g1t[1], g1t[2], g1t[3])
    a1t = jnp.stack(acc1, axis=0)
    g2t = _prep2_call(w2, b1r, a1t, g1t, dis)
    acc2 = _edge_call(src, dst, g2t[0], g2t[1], g2t[2], g2t[3])
    a2t = jnp.stack(acc2, axis=0)
    y2t = _out_call(b2r, a2t, g2t, dis)
    return y2t[:, :n][:, :, None]


# speedup vs baseline: 54.7226x; 1.0615x over previous
import jax
import jax.numpy as jnp
from jax import lax
from jax.experimental import pallas as pl
from jax.experimental.pallas import tpu as pltpu
from jax.experimental.pallas import tpu_sc as plsc

N_NODES = 100000
N_EDGES = 6400000
BATCH = 4

NC = 2
NS = 16
NW = NC * NS

N_PAD = 100352
ROWS_PER_TILE = N_PAD // NS

CHUNK = 8192
E_PER_W = 204800
N_CHUNKS = E_PER_W // CHUNK
E_PAD = E_PER_W * NW

_mesh = plsc.VectorSubcoreMesh(core_axis_name="c", subcore_axis_name="s")
_sc_params = pltpu.CompilerParams(use_tc_tiling_on_sc=False)


def _deg_body(dst_hbm, degp_hbm, idx_v, ones_v, fill_v, deg_sh):
    c = lax.axis_index("c")
    s = lax.axis_index("s")
    w = c * NS + s
    rs = pl.ds(s * ROWS_PER_TILE, ROWS_PER_TILE)

    def fill_ones(i, carry):
        ones_v[pl.ds(i * 16, 16)] = jnp.full((16,), 1.0, jnp.float32)
        return carry

    lax.fori_loop(0, CHUNK // 16, fill_ones, 0)

    def fill_half(i, carry):
        fill_v[pl.ds(i * 16, 16)] = jnp.full((16,), 0.5, jnp.float32)
        return carry

    lax.fori_loop(0, ROWS_PER_TILE // 16, fill_half, 0)
    pltpu.sync_copy(fill_v, deg_sh.at[rs])
    plsc.subcore_barrier()

    base = w * E_PER_W

    def body(i, carry):
        off = pl.multiple_of(base + i * CHUNK, CHUNK)
        pltpu.sync_copy(dst_hbm.at[pl.ds(off, CHUNK)], idx_v)
        pltpu.sync_copy(ones_v, deg_sh.at[idx_v], add=True)
        return carry

    lax.fori_loop(0, N_CHUNKS, body, 0)
    plsc.subcore_barrier()
    pltpu.sync_copy(deg_sh.at[rs], degp_hbm.at[c].at[rs])


_deg_call = pl.kernel(
    _deg_body,
    out_type=jax.ShapeDtypeStruct((NC, N_PAD), jnp.float32),
    mesh=_mesh,
    compiler_params=_sc_params,
    scratch_types=[
        pltpu.VMEM((CHUNK,), jnp.int32),
        pltpu.VMEM((CHUNK,), jnp.float32),
        pltpu.VMEM((ROWS_PER_TILE,), jnp.float32),
        pltpu.VMEM_SHARED((N_PAD,), jnp.float32),
    ],
)


def _edge_body(src_hbm, dst_hbm, g0_hbm, g1_hbm, g2_hbm, g3_hbm,
               a0_hbm, a1_hbm, a2_hbm, a3_hbm,
               src_v, dst_v, msg0_v, msg1_v, msg2_v, msg3_v, buf_v,
               g0_sh, g1_sh, g2_sh, g3_sh, ac0_sh, ac1_sh, ac2_sh, ac3_sh,
               isem0, isem1, gsem0, gsem1, gsem2, gsem3,
               ssem0, ssem1, ssem2, ssem3):
    c = lax.axis_index("c")
    s = lax.axis_index("s")
    w = c * NS + s
    rs = pl.ds(s * ROWS_PER_TILE, ROWS_PER_TILE)
    g_hbms = (g0_hbm, g1_hbm, g2_hbm, g3_hbm)
    a_hbms = (a0_hbm, a1_hbm, a2_hbm, a3_hbm)
    g_shs = (g0_sh, g1_sh, g2_sh, g3_sh)
    a_shs = (ac0_sh, ac1_sh, ac2_sh, ac3_sh)
    msgs = (msg0_v, msg1_v, msg2_v, msg3_v)
    gsems = (gsem0, gsem1, gsem2, gsem3)
    ssems = (ssem0, ssem1, ssem2, ssem3)

    for b in range(BATCH):
        pltpu.sync_copy(g_hbms[b].at[rs], buf_v)
        pltpu.sync_copy(buf_v, g_shs[b].at[rs])

    def fill_zero(i, carry):
        buf_v[pl.ds(i * 16, 16)] = jnp.full((16,), 0.0, jnp.float32)
        return carry

    lax.fori_loop(0, ROWS_PER_TILE // 16, fill_zero, 0)
    for b in range(BATCH):
        pltpu.sync_copy(buf_v, a_shs[b].at[rs])
    plsc.subcore_barrier()

    base = w * E_PER_W

    def body(i, carry):
        off = pl.multiple_of(base + i * CHUNK, CHUNK)
        di = pltpu.async_copy(src_hbm.at[pl.ds(off, CHUNK)], src_v, isem0)
        dj = pltpu.async_copy(dst_hbm.at[pl.ds(off, CHUNK)], dst_v, isem1)
        di.wait()
        gds = [pltpu.async_copy(g_shs[b].at[src_v], msgs[b], gsems[b])
               for b in range(BATCH)]
        dj.wait()
        sds = []
        for b in range(BATCH):
            gds[b].wait()
            sds.append(pltpu.async_copy(msgs[b], a_shs[b].at[dst_v],
                                        ssems[b], add=True))
        for b in range(BATCH):
            sds[b].wait()
        return carry

    lax.fori_loop(0, N_CHUNKS, body, 0)
    plsc.subcore_barrier()
    for b in range(BATCH):
        pltpu.sync_copy(a_shs[b].at[rs], a_hbms[b].at[c].at[rs])


_edge_call = pl.kernel(
    _edge_body,
    out_type=tuple(jax.ShapeDtypeStruct((NC, N_PAD), jnp.float32)
                   for _ in range(BATCH)),
    mesh=_mesh,
    compiler_params=_sc_params,
    scratch_types=(
        [pltpu.VMEM((CHUNK,), jnp.int32)] * 2
        + [pltpu.VMEM((CHUNK,), jnp.float32)] * 4
        + [pltpu.VMEM((ROWS_PER_TILE,), jnp.float32)]
        + [pltpu.VMEM_SHARED((N_PAD,), jnp.float32)] * 8
        + [pltpu.SemaphoreType.DMA] * 10
    ),
)


def _prep1_body(w1_ref, degp_ref, x4_ref, g1_ref, dis_ref):
    deg = degp_ref[0:1, :] + degp_ref[1:2, :]
    dis = lax.rsqrt(deg)
    dis_ref[...] = dis
    g1_ref[...] = x4_ref[...] * (dis * w1_ref[0:1, 0:1])


_prep1_call = pl.pallas_call(
    _prep1_body,
    out_shape=(
        jax.ShapeDtypeStruct((BATCH, N_PAD), jnp.float32),
        jax.ShapeDtypeStruct((1, N_PAD), jnp.float32),
    ),
)


def _prep2_body(w2_ref, b1_ref, at_ref, g1t_ref, dis_ref, g2_ref):
    a = at_ref[:, 0, :] + at_ref[:, 1, :]
    y1 = dis_ref[...] * (a + g1t_ref[...]) + b1_ref[0:1, 0:1]
    h = jnp.maximum(y1, 0.0)
    g2_ref[...] = h * (dis_ref[...] * w2_ref[0:1, 0:1])


_prep2_call = pl.pallas_call(
    _prep2_body,
    out_shape=jax.ShapeDtypeStruct((BATCH, N_PAD), jnp.float32),
)


def _out_body(b2_ref, at_ref, g2t_ref, dis_ref, y_ref):
    a = at_ref[:, 0, :] + at_ref[:, 1, :]
    y_ref[...] = dis_ref[...] * (a + g2t_ref[...]) + b2_ref[0:1, 0:1]


_out_call = pl.pallas_call(
    _out_body,
    out_shape=jax.ShapeDtypeStruct((BATCH, N_PAD), jnp.float32),
)


def kernel(x, edge_index, W1, b1, W2, b2):
    n = x.shape[1]
    e = edge_index.shape[1]
    e32 = edge_index.astype(jnp.int32)
    pad = jnp.full((2, E_PAD - e), n, dtype=jnp.int32)
    e32 = jnp.concatenate([e32, pad], axis=1)
    src = e32[0]
    dst = e32[1]

    x4 = jnp.pad(x[:, :, 0], ((0, 0), (0, N_PAD - n)))
    w1 = W1.reshape(1, 1)
    w2 = W2.reshape(1, 1)
    b1r = b1.reshape(1, 1)
    b2r = b2.reshape(1, 1)

    degp = _deg_call(dst)
    g1t, dis = _prep1_call(w1, degp, x4)
    acc1 = _edge_call(src, dst, g1t[0], g1t[1], g1t[2], g1t[3])
    a1t = jnp.stack(acc1, axis=0)
    g2t = _prep2_call(w2, b1r, a1t, g1t, dis)
    acc2 = _edge_call(src, dst, g2t[0], g2t[1], g2t[2], g2t[3])
    a2t = jnp.stack(acc2, axis=0)
    y2t = _out_call(b2r, a2t, g2t, dis)
    return y2t[:, :n][:, :, None]
